# baseline (speedup 1.0000x reference)
import jax
import jax.numpy as jnp
from jax import lax
from jax.experimental import pallas as pl
from jax.experimental.pallas import tpu as pltpu
from jax.experimental.pallas import tpu_sc as plsc

N_ENTITIES = 100000
N_RELATIONS = 500
EMBED_DIM = 64
BATCH = 1024

_NC = 2
_NS = 16
_NL = 16
_NW = _NC * _NS
_B_PER_W = BATCH // _NW


def _sc_gather_body(heads_hbm, rels_hbm, ent_hbm, rel_hbm, out_hbm,
                    hidx_v, ridx_v, e_v, r_v, sem):
    wid = lax.axis_index("s") * _NC + lax.axis_index("c")
    base = wid * _B_PER_W
    pltpu.sync_copy(heads_hbm.at[pl.ds(base, _B_PER_W)], hidx_v)
    pltpu.sync_copy(rels_hbm.at[pl.ds(base, _B_PER_W)], ridx_v)
    cp_e = pltpu.async_copy(ent_hbm.at[hidx_v], e_v, sem)
    cp_r = pltpu.async_copy(rel_hbm.at[ridx_v], r_v, sem)
    cp_e.wait()
    cp_r.wait()
    for i in range(_B_PER_W):
        for c in range(EMBED_DIM // _NL):
            sl = pl.ds(c * _NL, _NL)
            e_v[i, sl] = e_v[i, sl] + r_v[i, sl]
    pltpu.sync_copy(e_v, out_hbm.at[pl.ds(base, _B_PER_W)])


def _sc_gather_hr(heads, relations, entity_emb, relation_emb):
    mesh = plsc.VectorSubcoreMesh(core_axis_name="c", subcore_axis_name="s")
    fn = pl.kernel(
        _sc_gather_body, mesh=mesh,
        compiler_params=pltpu.CompilerParams(use_tc_tiling_on_sc=False),
        out_type=jax.ShapeDtypeStruct((BATCH, EMBED_DIM), jnp.float32),
        scratch_types=[
            pltpu.VMEM((_B_PER_W,), jnp.int32),
            pltpu.VMEM((_B_PER_W,), jnp.int32),
            pltpu.VMEM((_B_PER_W, EMBED_DIM), jnp.float32),
            pltpu.VMEM((_B_PER_W, EMBED_DIM), jnp.float32),
            pltpu.SemaphoreType.DMA,
        ],
    )
    return fn(heads, relations, entity_emb, relation_emb)


_RB = 32
_NRO = BATCH // _RB
_DA = EMBED_DIM + 2
_TSQ_CH = 12544


def _tc_score_body(hr_ref, et_hbm, out_ref, pred_ref, a_sc, bt_sc, sem):
    ro = pl.program_id(0)

    @pl.when(ro == 0)
    def _():
        pltpu.make_async_copy(
            et_hbm, bt_sc.at[pl.ds(0, EMBED_DIM), :], sem).start()
        pltpu.make_async_copy(
            et_hbm, bt_sc.at[pl.ds(0, EMBED_DIM), :], sem).wait()
        for c in range((N_ENTITIES + _TSQ_CH - 1) // _TSQ_CH):
            w = min(_TSQ_CH, N_ENTITIES - c * _TSQ_CH)
            sl = pl.ds(c * _TSQ_CH, w)
            e = bt_sc[pl.ds(0, EMBED_DIM), sl]
            bt_sc[pl.ds(EMBED_DIM, 1), sl] = jnp.sum(e * e, axis=0,
                                                     keepdims=True)
            bt_sc[pl.ds(EMBED_DIM + 1, 1), sl] = jnp.full((1, w), 1.0,
                                                          jnp.float32)

    hr = hr_ref[...]
    a_sc[:, 0:EMBED_DIM] = 2.0 * hr
    a_sc[:, EMBED_DIM:EMBED_DIM + 1] = jnp.full((_RB, 1), -1.0, jnp.float32)
    a_sc[:, EMBED_DIM + 1:_DA] = -jnp.sum(hr * hr, axis=1, keepdims=True)

    scores = lax.dot_general(a_sc[...], bt_sc[...], (((1,), (0,)), ((), ())),
                             preferred_element_type=jnp.float32)
    out_ref[...] = scores

    col = lax.broadcasted_iota(jnp.int32, (_RB, N_ENTITIES), 1)
    lm = jnp.max(scores, axis=1, keepdims=True)
    la = jnp.min(jnp.where(scores == lm, col, jnp.int32(2**31 - 1)),
                 axis=1, keepdims=True)
    pred_ref[...] = la


def _tc_score(hr, entity_t):
    return pl.pallas_call(
        _tc_score_body,
        grid=(_NRO,),
        compiler_params=pltpu.CompilerParams(
            vmem_limit_bytes=63 * 1024 * 1024),
        in_specs=[
            pl.BlockSpec((_RB, EMBED_DIM), lambda ro: (ro, 0)),
            pl.BlockSpec(memory_space=pltpu.MemorySpace.HBM),
        ],
        out_specs=(
            pl.BlockSpec((_RB, N_ENTITIES), lambda ro: (ro, 0)),
            pl.BlockSpec((_RB, 1), lambda ro: (ro, 0)),
        ),
        out_shape=(
            jax.ShapeDtypeStruct((BATCH, N_ENTITIES), jnp.float32),
            jax.ShapeDtypeStruct((BATCH, 1), jnp.int32),
        ),
        scratch_shapes=[
            pltpu.VMEM((_RB, _DA), jnp.float32),
            pltpu.VMEM((_DA, N_ENTITIES), jnp.float32),
            pltpu.SemaphoreType.DMA,
        ],
    )(hr, entity_t)


def kernel(queries, entity_emb, relation_emb):
    heads = queries[:, 0].astype(jnp.int32)
    relations = queries[:, 1].astype(jnp.int32)
    hr = _sc_gather_hr(heads, relations, entity_emb, relation_emb)
    entity_t = entity_emb.T
    all_scores, pred = _tc_score(hr, entity_t)
    return all_scores, pred.reshape(BATCH)


# speedup vs baseline: 1.1135x; 1.1135x over previous
import jax
import jax.numpy as jnp
from jax import lax
from jax.experimental import pallas as pl
from jax.experimental.pallas import tpu as pltpu
from jax.experimental.pallas import tpu_sc as plsc

N_ENTITIES = 100000
N_RELATIONS = 500
EMBED_DIM = 64
BATCH = 1024

_NC = 2
_NS = 16
_NL = 16
_NW = _NC * _NS
_B_PER_W = BATCH // _NW


def _sc_gather_body(heads_hbm, rels_hbm, ent_hbm, rel_hbm, out_hbm,
                    hidx_v, ridx_v, e_v, r_v, sem):
    wid = lax.axis_index("s") * _NC + lax.axis_index("c")
    base = wid * _B_PER_W
    pltpu.sync_copy(heads_hbm.at[pl.ds(base, _B_PER_W)], hidx_v)
    pltpu.sync_copy(rels_hbm.at[pl.ds(base, _B_PER_W)], ridx_v)
    cp_e = pltpu.async_copy(ent_hbm.at[hidx_v], e_v, sem)
    cp_r = pltpu.async_copy(rel_hbm.at[ridx_v], r_v, sem)
    cp_e.wait()
    cp_r.wait()
    for i in range(_B_PER_W):
        for c in range(EMBED_DIM // _NL):
            sl = pl.ds(c * _NL, _NL)
            e_v[i, sl] = e_v[i, sl] + r_v[i, sl]
    pltpu.sync_copy(e_v, out_hbm.at[pl.ds(base, _B_PER_W)])


def _sc_gather_hr(heads, relations, entity_emb, relation_emb):
    mesh = plsc.VectorSubcoreMesh(core_axis_name="c", subcore_axis_name="s")
    fn = pl.kernel(
        _sc_gather_body, mesh=mesh,
        compiler_params=pltpu.CompilerParams(use_tc_tiling_on_sc=False),
        out_type=jax.ShapeDtypeStruct((BATCH, EMBED_DIM), jnp.float32),
        scratch_types=[
            pltpu.VMEM((_B_PER_W,), jnp.int32),
            pltpu.VMEM((_B_PER_W,), jnp.int32),
            pltpu.VMEM((_B_PER_W, EMBED_DIM), jnp.float32),
            pltpu.VMEM((_B_PER_W, EMBED_DIM), jnp.float32),
            pltpu.SemaphoreType.DMA,
        ],
    )
    return fn(heads, relations, entity_emb, relation_emb)


_TN = 2048
_NBLK = (N_ENTITIES + _TN - 1) // _TN
_DA = EMBED_DIM + 2


def _tc_score_body(hr_ref, e_ref, outT_ref, pred_ref,
                   a_sc, b_sc, best_val, best_idx):
    j = pl.program_id(0)

    @pl.when(j == 0)
    def _():
        hr = hr_ref[...]
        a_sc[:, 0:EMBED_DIM] = 2.0 * hr
        a_sc[:, EMBED_DIM:EMBED_DIM + 1] = jnp.full((BATCH, 1), -1.0,
                                                    jnp.float32)
        a_sc[:, EMBED_DIM + 1:_DA] = -jnp.sum(hr * hr, axis=1, keepdims=True)
        best_val[...] = jnp.full((8, BATCH), -jnp.inf, jnp.float32)
        best_idx[...] = jnp.zeros((8, BATCH), jnp.int32)

    e = e_ref[...]
    b_sc[:, 0:EMBED_DIM] = e
    b_sc[:, EMBED_DIM:EMBED_DIM + 1] = jnp.sum(e * e, axis=1, keepdims=True)
    b_sc[:, EMBED_DIM + 1:_DA] = jnp.full((_TN, 1), 1.0, jnp.float32)

    scoresT = lax.dot_general(b_sc[...], a_sc[...], (((1,), (1,)), ((), ())),
                              precision=lax.Precision.HIGHEST,
                              preferred_element_type=jnp.float32)
    outT_ref[...] = scoresT

    row = j * _TN + lax.broadcasted_iota(jnp.int32, (_TN, BATCH), 0)
    big = jnp.int32(2**31 - 1)

    def _fold(local_max, local_arg):
        lm8 = jnp.broadcast_to(local_max, (8, BATCH))
        la8 = jnp.broadcast_to(local_arg, (8, BATCH))
        better = lm8 > best_val[...]
        best_val[...] = jnp.where(better, lm8, best_val[...])
        best_idx[...] = jnp.where(better, la8, best_idx[...])

    @pl.when(j < _NBLK - 1)
    def _():
        lm = jnp.max(scoresT, axis=0, keepdims=True)
        la = jnp.min(jnp.where(scoresT == lm, row, big), axis=0,
                     keepdims=True)
        _fold(lm, la)

    @pl.when(j == _NBLK - 1)
    def _():
        valid = row < N_ENTITIES
        s_m = jnp.where(valid, scoresT, -jnp.inf)
        lm = jnp.max(s_m, axis=0, keepdims=True)
        la = jnp.min(jnp.where(s_m == lm, row, big), axis=0, keepdims=True)
        _fold(lm, la)

    pred_ref[...] = best_idx[...]


def _tc_score(hr, entity_emb):
    return pl.pallas_call(
        _tc_score_body,
        grid=(_NBLK,),
        compiler_params=pltpu.CompilerParams(
            vmem_limit_bytes=60 * 1024 * 1024),
        in_specs=[
            pl.BlockSpec((BATCH, EMBED_DIM), lambda j: (0, 0)),
            pl.BlockSpec((_TN, EMBED_DIM), lambda j: (j, 0)),
        ],
        out_specs=(
            pl.BlockSpec((_TN, BATCH), lambda j: (j, 0)),
            pl.BlockSpec((8, BATCH), lambda j: (0, 0)),
        ),
        out_shape=(
            jax.ShapeDtypeStruct((N_ENTITIES, BATCH), jnp.float32),
            jax.ShapeDtypeStruct((8, BATCH), jnp.int32),
        ),
        scratch_shapes=[
            pltpu.VMEM((BATCH, _DA), jnp.float32),
            pltpu.VMEM((_TN, _DA), jnp.float32),
            pltpu.VMEM((8, BATCH), jnp.float32),
            pltpu.VMEM((8, BATCH), jnp.int32),
        ],
    )(hr, entity_emb)


def kernel(queries, entity_emb, relation_emb):
    heads = queries[:, 0].astype(jnp.int32)
    relations = queries[:, 1].astype(jnp.int32)
    hr = _sc_gather_hr(heads, relations, entity_emb, relation_emb)
    scores_t, pred8 = _tc_score(hr, entity_emb)
    return scores_t.T, pred8[0]
